# Initial kernel scaffold; baseline (speedup 1.0000x reference)
import jax
import jax.numpy as jnp
from jax.experimental import pallas as pl


def kernel(x, seg, tok_embed, pos_embed, seg_embed):
    raise NotImplementedError("write your pallas kernel here")


# speedup vs baseline: 5.0515x; 5.0515x over previous
import functools

import jax
import jax.numpy as jnp
from jax import lax
from jax.experimental import pallas as pl
from jax.experimental.pallas import tpu as pltpu
from jax.experimental.pallas import tpu_sc as plsc

D = 128
MAXLEN = 512
NSEG = 2
NC = 2
NS = 16
NW = NC * NS
CHUNK = 128
LANES = 16


def _psum_body(pos_ref, seg_ref, out_ref):
    p = pos_ref[...]
    out_ref[0:MAXLEN, :] = p + seg_ref[0:1, :]
    out_ref[MAXLEN : 2 * MAXLEN, :] = p + seg_ref[1:2, :]


def _build_psum(pos_embed, seg_embed):
    return pl.pallas_call(
        _psum_body,
        out_shape=jax.ShapeDtypeStruct((MAXLEN * NSEG, D), jnp.float32),
    )(pos_embed, seg_embed)


def _sc_body(tok_hbm, psum_hbm, x_hbm, seg_hbm, out_hbm,
             xbuf, sbuf, cidx, tokbuf, psbuf, sem0, sem1):
    rows = out_hbm.shape[0]
    rows_per_w = rows // NW
    nchunk = rows_per_w // CHUNK
    w = lax.axis_index("s") * NC + lax.axis_index("c")
    iota = lax.iota(jnp.int32, LANES)

    def chunk_body(c, carry):
        base = w * rows_per_w + c * CHUNK
        pltpu.sync_copy(x_hbm.at[pl.ds(base, CHUNK)], xbuf)
        pltpu.sync_copy(seg_hbm.at[pl.ds(base, CHUNK)], sbuf)
        posbase = lax.rem(c, MAXLEN // CHUNK) * CHUNK
        for i in range(CHUNK // LANES):
            sl = pl.ds(i * LANES, LANES)
            cidx[sl] = sbuf[sl] * MAXLEN + (iota + (i * LANES + posbase))
        cp0 = pltpu.async_copy(tok_hbm.at[xbuf], tokbuf, sem0)
        cp1 = pltpu.async_copy(psum_hbm.at[cidx], psbuf, sem1)
        cp0.wait()
        cp1.wait()

        def row_body(r, rc):
            for j in range(D // LANES):
                sl = pl.ds(j * LANES, LANES)
                tokbuf[r, sl] = tokbuf[r, sl] + psbuf[r, sl]
            return rc

        lax.fori_loop(0, CHUNK, row_body, 0, unroll=2)
        pltpu.sync_copy(tokbuf, out_hbm.at[pl.ds(base, CHUNK)])
        return carry

    lax.fori_loop(0, nchunk, chunk_body, 0)


def _sc_gather(tok_embed, psum, x_flat, seg_flat):
    rows = x_flat.shape[0]
    fn = functools.partial(
        pl.kernel,
        out_type=jax.ShapeDtypeStruct((rows, D), jnp.float32),
        mesh=plsc.VectorSubcoreMesh(core_axis_name="c", subcore_axis_name="s"),
        scratch_types=[
            pltpu.VMEM((CHUNK,), jnp.int32),
            pltpu.VMEM((CHUNK,), jnp.int32),
            pltpu.VMEM((CHUNK,), jnp.int32),
            pltpu.VMEM((CHUNK, D), jnp.float32),
            pltpu.VMEM((CHUNK, D), jnp.float32),
            pltpu.SemaphoreType.DMA,
            pltpu.SemaphoreType.DMA,
        ],
    )(_sc_body)
    return fn(tok_embed, psum, x_flat, seg_flat)


def kernel(x, seg, tok_embed, pos_embed, seg_embed):
    batch, seqlen = x.shape
    x_flat = x.reshape(-1).astype(jnp.int32)
    seg_flat = seg.reshape(-1).astype(jnp.int32)
    psum = _build_psum(pos_embed, seg_embed)
    out = _sc_gather(tok_embed, psum, x_flat, seg_flat)
    return out.reshape(batch, seqlen, D)


# speedup vs baseline: 8.2478x; 1.6327x over previous
import functools

import jax
import jax.numpy as jnp
from jax import lax
from jax.experimental import pallas as pl
from jax.experimental.pallas import tpu as pltpu
from jax.experimental.pallas import tpu_sc as plsc

D = 128
MAXLEN = 512
NSEG = 2
NC = 2
NS = 16
NW = NC * NS
CHUNK = 128
LANES = 16
SUB = 4
SUBROWS = CHUNK // SUB


def _psum_body(pos_ref, seg_ref, out_ref):
    p = pos_ref[...]
    out_ref[0:MAXLEN, :] = p + seg_ref[0:1, :]
    out_ref[MAXLEN : 2 * MAXLEN, :] = p + seg_ref[1:2, :]


def _build_psum(pos_embed, seg_embed):
    return pl.pallas_call(
        _psum_body,
        out_shape=jax.ShapeDtypeStruct((MAXLEN * NSEG, D), jnp.float32),
    )(pos_embed, seg_embed)


def _sc_body(tok_hbm, psum_hbm, x_hbm, seg_hbm, out_hbm,
             xall, call, tok0, ps0, tok1, ps1,
             semt0, semp0, semo0, semt1, semp1, semo1):
    nchunk_w = xall.shape[0]
    rows_per_w = nchunk_w * CHUNK
    w = lax.axis_index("s") * NC + lax.axis_index("c")
    iota = lax.iota(jnp.int32, LANES)
    toks = (tok0, tok1)
    pss = (ps0, ps1)
    semts = (semt0, semt1)
    semps = (semp0, semp1)
    semos = (semo0, semo1)

    pltpu.sync_copy(x_hbm.at[pl.ds(w * nchunk_w, nchunk_w)], xall)
    pltpu.sync_copy(seg_hbm.at[pl.ds(w * nchunk_w, nchunk_w)], call)

    def cidx_body(j, carry):
        posbase = lax.rem(j, MAXLEN // CHUNK) * CHUNK
        for i in range(CHUNK // LANES):
            sl = pl.ds(i * LANES, LANES)
            call[j, sl] = call[j, sl] * MAXLEN + (iota + (i * LANES + posbase))
        return carry

    lax.fori_loop(0, nchunk_w, cidx_body, 0)

    def fire_gathers(c, b):
        cpt = pltpu.async_copy(tok_hbm.at[xall.at[c]], toks[b], semts[b])
        cpp = pltpu.async_copy(psum_hbm.at[call.at[c]], pss[b], semps[b])
        return cpt, cpp

    def turn(c, b, refire):
        pltpu.make_async_copy(tok_hbm.at[xall.at[c]], toks[b], semts[b]).wait()
        pltpu.make_async_copy(psum_hbm.at[call.at[c]], pss[b], semps[b]).wait()
        rowbase = (w * nchunk_w + c) * CHUNK
        out_handles = []
        for q in range(SUB):
            def add_body(r, carry):
                for j in range(D // LANES):
                    sl = pl.ds(j * LANES, LANES)
                    pss[b][r, sl] = toks[b][r, sl] + pss[b][r, sl]
                return carry

            lax.fori_loop(q * SUBROWS, (q + 1) * SUBROWS, add_body, 0, unroll=2)
            out_handles.append(pltpu.async_copy(
                pss[b].at[pl.ds(q * SUBROWS, SUBROWS)],
                out_hbm.at[pl.ds(rowbase + q * SUBROWS, SUBROWS)],
                semos[b]))
        if refire:
            pltpu.async_copy(tok_hbm.at[xall.at[c + 2]], toks[b], semts[b])
        for h in out_handles:
            h.wait()
        if refire:
            pltpu.async_copy(psum_hbm.at[call.at[c + 2]], pss[b], semps[b])

    fire_gathers(0, 0)
    fire_gathers(1, 1)

    def main_body(cc, carry):
        for b in range(2):
            turn(2 * cc + b, b, refire=True)
        return carry

    lax.fori_loop(0, nchunk_w // 2 - 1, main_body, 0)
    turn(nchunk_w - 2, 0, refire=False)
    turn(nchunk_w - 1, 1, refire=False)


def _sc_gather(tok_embed, psum, x_blk, seg_blk):
    nblk = x_blk.shape[0]
    rows = nblk * CHUNK
    fn = functools.partial(
        pl.kernel,
        out_type=jax.ShapeDtypeStruct((rows, D), jnp.float32),
        mesh=plsc.VectorSubcoreMesh(core_axis_name="c", subcore_axis_name="s"),
        scratch_types=[
            pltpu.VMEM((nblk // NW, CHUNK), jnp.int32),
            pltpu.VMEM((nblk // NW, CHUNK), jnp.int32),
            pltpu.VMEM((CHUNK, D), jnp.float32),
            pltpu.VMEM((CHUNK, D), jnp.float32),
            pltpu.VMEM((CHUNK, D), jnp.float32),
            pltpu.VMEM((CHUNK, D), jnp.float32),
            pltpu.SemaphoreType.DMA,
            pltpu.SemaphoreType.DMA,
            pltpu.SemaphoreType.DMA,
            pltpu.SemaphoreType.DMA,
            pltpu.SemaphoreType.DMA,
            pltpu.SemaphoreType.DMA,
        ],
    )(_sc_body)
    return fn(tok_embed, psum, x_blk, seg_blk)


def kernel(x, seg, tok_embed, pos_embed, seg_embed):
    batch, seqlen = x.shape
    x_blk = x.reshape(-1, CHUNK).astype(jnp.int32)
    seg_blk = seg.reshape(-1, CHUNK).astype(jnp.int32)
    psum = _build_psum(pos_embed, seg_embed)
    out = _sc_gather(tok_embed, psum, x_blk, seg_blk)
    return out.reshape(batch, seqlen, D)


# speedup vs baseline: 8.2824x; 1.0042x over previous
import functools

import jax
import jax.numpy as jnp
from jax import lax
from jax.experimental import pallas as pl
from jax.experimental.pallas import tpu as pltpu
from jax.experimental.pallas import tpu_sc as plsc

D = 128
MAXLEN = 512
NSEG = 2
NC = 2
NS = 16
NW = NC * NS
CHUNK = 128
LANES = 16
SUB = 4
SUBROWS = CHUNK // SUB


def _sc_body(tok_hbm, pos_hbm, seg_emb_hbm, x_hbm, seg_hbm, out_hbm,
             psum_shr, xall, call, tok0, ps0, tok1, ps1,
             semt0, semp0, semo0, semt1, semp1, semo1):
    nchunk_w = xall.shape[0]
    rows_per_w = nchunk_w * CHUNK
    w = lax.axis_index("s") * NC + lax.axis_index("c")
    iota = lax.iota(jnp.int32, LANES)
    toks = (tok0, tok1)
    pss = (ps0, ps1)
    semts = (semt0, semt1)
    semps = (semp0, semp1)
    semos = (semo0, semo1)

    sid = lax.axis_index("s")
    prows = (MAXLEN * NSEG) // NS
    g = sid // (MAXLEN // prows)
    s0 = lax.rem(sid * prows, MAXLEN)
    pltpu.sync_copy(seg_emb_hbm, tok0.at[pl.ds(0, NSEG)])
    pltpu.sync_copy(pos_hbm.at[pl.ds(s0, prows)], ps0.at[pl.ds(0, prows)])

    def prow_body(r, carry):
        for j in range(D // LANES):
            sl = pl.ds(j * LANES, LANES)
            ps0[r, sl] = ps0[r, sl] + tok0[g, sl]
        return carry

    lax.fori_loop(0, prows, prow_body, 0, unroll=2)
    pltpu.sync_copy(ps0.at[pl.ds(0, prows)],
                    psum_shr.at[pl.ds(sid * prows, prows)])
    plsc.subcore_barrier()

    pltpu.sync_copy(x_hbm.at[pl.ds(w * nchunk_w, nchunk_w)], xall)
    pltpu.sync_copy(seg_hbm.at[pl.ds(w * nchunk_w, nchunk_w)], call)

    def cidx_body(j, carry):
        posbase = lax.rem(j, MAXLEN // CHUNK) * CHUNK
        for i in range(CHUNK // LANES):
            sl = pl.ds(i * LANES, LANES)
            call[j, sl] = call[j, sl] * MAXLEN + (iota + (i * LANES + posbase))
        return carry

    lax.fori_loop(0, nchunk_w, cidx_body, 0)

    def fire_gathers(c, b):
        cpt = pltpu.async_copy(tok_hbm.at[xall.at[c]], toks[b], semts[b])
        cpp = pltpu.async_copy(psum_shr.at[call.at[c]], pss[b], semps[b])
        return cpt, cpp

    def turn(c, b, refire):
        pltpu.make_async_copy(tok_hbm.at[xall.at[c]], toks[b], semts[b]).wait()
        pltpu.make_async_copy(psum_shr.at[call.at[c]], pss[b], semps[b]).wait()
        rowbase = (w * nchunk_w + c) * CHUNK
        out_handles = []
        for q in range(SUB):
            def add_body(r, carry):
                for j in range(D // LANES):
                    sl = pl.ds(j * LANES, LANES)
                    pss[b][r, sl] = toks[b][r, sl] + pss[b][r, sl]
                return carry

            lax.fori_loop(q * SUBROWS, (q + 1) * SUBROWS, add_body, 0, unroll=2)
            out_handles.append(pltpu.async_copy(
                pss[b].at[pl.ds(q * SUBROWS, SUBROWS)],
                out_hbm.at[pl.ds(rowbase + q * SUBROWS, SUBROWS)],
                semos[b]))
        if refire:
            pltpu.async_copy(tok_hbm.at[xall.at[c + 2]], toks[b], semts[b])
        for h in out_handles:
            h.wait()
        if refire:
            pltpu.async_copy(psum_shr.at[call.at[c + 2]], pss[b], semps[b])

    fire_gathers(0, 0)
    fire_gathers(1, 1)

    def main_body(cc, carry):
        for b in range(2):
            turn(2 * cc + b, b, refire=True)
        return carry

    lax.fori_loop(0, nchunk_w // 2 - 1, main_body, 0)
    turn(nchunk_w - 2, 0, refire=False)
    turn(nchunk_w - 1, 1, refire=False)


def _sc_gather(tok_embed, pos_embed, seg_embed, x_blk, seg_blk):
    nblk = x_blk.shape[0]
    rows = nblk * CHUNK
    fn = functools.partial(
        pl.kernel,
        out_type=jax.ShapeDtypeStruct((rows, D), jnp.float32),
        mesh=plsc.VectorSubcoreMesh(core_axis_name="c", subcore_axis_name="s"),
        scratch_types=[
            pltpu.VMEM_SHARED((MAXLEN * NSEG, D), jnp.float32),
            pltpu.VMEM((nblk // NW, CHUNK), jnp.int32),
            pltpu.VMEM((nblk // NW, CHUNK), jnp.int32),
            pltpu.VMEM((CHUNK, D), jnp.float32),
            pltpu.VMEM((CHUNK, D), jnp.float32),
            pltpu.VMEM((CHUNK, D), jnp.float32),
            pltpu.VMEM((CHUNK, D), jnp.float32),
            pltpu.SemaphoreType.DMA,
            pltpu.SemaphoreType.DMA,
            pltpu.SemaphoreType.DMA,
            pltpu.SemaphoreType.DMA,
            pltpu.SemaphoreType.DMA,
            pltpu.SemaphoreType.DMA,
        ],
    )(_sc_body)
    return fn(tok_embed, pos_embed, seg_embed, x_blk, seg_blk)


def kernel(x, seg, tok_embed, pos_embed, seg_embed):
    batch, seqlen = x.shape
    x_blk = x.reshape(-1, CHUNK).astype(jnp.int32)
    seg_blk = seg.reshape(-1, CHUNK).astype(jnp.int32)
    out = _sc_gather(tok_embed, pos_embed, seg_embed, x_blk, seg_blk)
    return out.reshape(batch, seqlen, D)


# speedup vs baseline: 23.3260x; 2.8163x over previous
import functools

import jax
import jax.numpy as jnp
from jax import lax
from jax.experimental import pallas as pl
from jax.experimental.pallas import tpu as pltpu
from jax.experimental.pallas import tpu_sc as plsc

D = 128
MAXLEN = 512
NSEG = 2
NC = 2
NS = 16
NW = NC * NS
CHUNK = 128
LANES = 16
SUB = 4
SUBROWS = CHUNK // SUB


def _sc_body(tok_hbm, pos_hbm, seg_emb_hbm, x_hbm, seg_hbm, out_hbm,
             psum_shr, xall, call, ident, tok0, ps0, tok1, ps1,
             semt0, semp0, semo0, semt1, semp1, semo1):
    nchunk_w = xall.shape[0]
    rows_per_w = nchunk_w * CHUNK
    w = lax.axis_index("s") * NC + lax.axis_index("c")
    iota = lax.iota(jnp.int32, LANES)
    toks = (tok0, tok1)
    pss = (ps0, ps1)
    semts = (semt0, semt1)
    semps = (semp0, semp1)
    semos = (semo0, semo1)

    sid = lax.axis_index("s")
    prows = (MAXLEN * NSEG) // NS
    g = sid // (MAXLEN // prows)
    s0 = lax.rem(sid * prows, MAXLEN)
    pltpu.sync_copy(seg_emb_hbm, tok0.at[pl.ds(0, NSEG)])
    pltpu.sync_copy(pos_hbm.at[pl.ds(s0, prows)], ps0.at[pl.ds(0, prows)])

    def prow_body(r, carry):
        for j in range(D // LANES):
            sl = pl.ds(j * LANES, LANES)
            ps0[r, sl] = ps0[r, sl] + tok0[g, sl]
        return carry

    lax.fori_loop(0, prows, prow_body, 0, unroll=2)
    pltpu.sync_copy(ps0.at[pl.ds(0, prows)],
                    psum_shr.at[pl.ds(sid * prows, prows)])
    plsc.subcore_barrier()

    pltpu.sync_copy(x_hbm.at[pl.ds(w * nchunk_w, nchunk_w)], xall)
    pltpu.sync_copy(seg_hbm.at[pl.ds(w * nchunk_w, nchunk_w)], call)

    def cidx_body(j, carry):
        posbase = lax.rem(j, MAXLEN // CHUNK) * CHUNK
        for i in range(CHUNK // LANES):
            sl = pl.ds(i * LANES, LANES)
            call[j, sl] = call[j, sl] * MAXLEN + (iota + (i * LANES + posbase))
        return carry

    lax.fori_loop(0, nchunk_w, cidx_body, 0)

    for q in range(SUB):
        for i in range(SUBROWS // LANES):
            ident[q, pl.ds(i * LANES, LANES)] = iota + (q * SUBROWS + i * LANES)

    def fire_gathers(c, b):
        cpt = pltpu.async_copy(tok_hbm.at[xall.at[c]], toks[b], semts[b])
        cpp = pltpu.async_copy(psum_shr.at[call.at[c]], pss[b], semps[b])
        return cpt, cpp

    def turn(c, b, refire):
        pltpu.make_async_copy(tok_hbm.at[xall.at[c]], toks[b], semts[b]).wait()
        pltpu.make_async_copy(psum_shr.at[call.at[c]], pss[b], semps[b]).wait()
        rowbase = (w * nchunk_w + c) * CHUNK
        out_handles = []
        for q in range(SUB):
            @plsc.parallel_loop(q * SUBROWS, (q + 1) * SUBROWS, unroll=4)
            def add_body(r):
                for j in range(D // LANES):
                    sl = pl.ds(j * LANES, LANES)
                    pss[b][r, sl] = toks[b][r, sl] + pss[b][r, sl]

            out_handles.append(pltpu.async_copy(
                pss[b].at[pl.ds(q * SUBROWS, SUBROWS)],
                out_hbm.at[pl.ds(rowbase + q * SUBROWS, SUBROWS)],
                semos[b]))
        if refire:
            pltpu.async_copy(tok_hbm.at[xall.at[c + 2]], toks[b], semts[b])
        for h in out_handles:
            h.wait()
        if refire:
            pltpu.async_copy(psum_shr.at[call.at[c + 2]], pss[b], semps[b])

    fire_gathers(0, 0)
    fire_gathers(1, 1)

    def main_body(cc, carry):
        for b in range(2):
            turn(2 * cc + b, b, refire=True)
        return carry

    lax.fori_loop(0, nchunk_w // 2 - 1, main_body, 0)
    turn(nchunk_w - 2, 0, refire=False)
    turn(nchunk_w - 1, 1, refire=False)


def _sc_gather(tok_embed, pos_embed, seg_embed, x_blk, seg_blk):
    nblk = x_blk.shape[0]
    rows = nblk * CHUNK
    fn = functools.partial(
        pl.kernel,
        out_type=jax.ShapeDtypeStruct((rows, D), jnp.float32),
        mesh=plsc.VectorSubcoreMesh(core_axis_name="c", subcore_axis_name="s"),
        scratch_types=[
            pltpu.VMEM_SHARED((MAXLEN * NSEG, D), jnp.float32),
            pltpu.VMEM((nblk // NW, CHUNK), jnp.int32),
            pltpu.VMEM((nblk // NW, CHUNK), jnp.int32),
            pltpu.VMEM((SUB, SUBROWS), jnp.int32),
            pltpu.VMEM((CHUNK, D), jnp.float32),
            pltpu.VMEM((CHUNK, D), jnp.float32),
            pltpu.VMEM((CHUNK, D), jnp.float32),
            pltpu.VMEM((CHUNK, D), jnp.float32),
            pltpu.SemaphoreType.DMA,
            pltpu.SemaphoreType.DMA,
            pltpu.SemaphoreType.DMA,
            pltpu.SemaphoreType.DMA,
            pltpu.SemaphoreType.DMA,
            pltpu.SemaphoreType.DMA,
        ],
    )(_sc_body)
    return fn(tok_embed, pos_embed, seg_embed, x_blk, seg_blk)


def kernel(x, seg, tok_embed, pos_embed, seg_embed):
    batch, seqlen = x.shape
    x_blk = x.reshape(-1, CHUNK).astype(jnp.int32)
    seg_blk = seg.reshape(-1, CHUNK).astype(jnp.int32)
    out = _sc_gather(tok_embed, pos_embed, seg_embed, x_blk, seg_blk)
    return out.reshape(batch, seqlen, D)


# speedup vs baseline: 23.3962x; 1.0030x over previous
import functools

import jax
import jax.numpy as jnp
from jax import lax
from jax.experimental import pallas as pl
from jax.experimental.pallas import tpu as pltpu
from jax.experimental.pallas import tpu_sc as plsc

D = 128
MAXLEN = 512
NSEG = 2
NC = 2
NS = 16
NW = NC * NS
CHUNK = 128
LANES = 16
SUB = 4
SUBROWS = CHUNK // SUB


def _sc_body(tok_hbm, pos_hbm, seg_emb_hbm, x_hbm, seg_hbm, out_hbm,
             psum_shr, xall, call, ident, tok0, ps0, tok1, ps1,
             semt0, semp0, semo0, semt1, semp1, semo1):
    nchunk_w = xall.shape[0]
    rows_per_w = nchunk_w * CHUNK
    w = lax.axis_index("s") * NC + lax.axis_index("c")
    iota = lax.iota(jnp.int32, LANES)
    toks = (tok0, tok1)
    pss = (ps0, ps1)
    semts = (semt0, semt1)
    semps = (semp0, semp1)
    semos = (semo0, semo1)

    sid = lax.axis_index("s")
    prows = (MAXLEN * NSEG) // NS
    g = sid // (MAXLEN // prows)
    s0 = lax.rem(sid * prows, MAXLEN)
    pltpu.sync_copy(seg_emb_hbm, tok0.at[pl.ds(0, NSEG)])
    pltpu.sync_copy(pos_hbm.at[pl.ds(s0, prows)], ps0.at[pl.ds(0, prows)])

    def prow_body(r, carry):
        for j in range(D // LANES):
            sl = pl.ds(j * LANES, LANES)
            ps0[r, sl] = ps0[r, sl] + tok0[g, sl]
        return carry

    lax.fori_loop(0, prows, prow_body, 0, unroll=2)
    pltpu.sync_copy(ps0.at[pl.ds(0, prows)],
                    psum_shr.at[pl.ds(sid * prows, prows)])
    plsc.subcore_barrier()

    pltpu.sync_copy(x_hbm.at[pl.ds(w * nchunk_w, nchunk_w)], xall)
    pltpu.sync_copy(seg_hbm.at[pl.ds(w * nchunk_w, nchunk_w)], call)

    def cidx_body(j, carry):
        posbase = lax.rem(j, MAXLEN // CHUNK) * CHUNK
        for i in range(CHUNK // LANES):
            sl = pl.ds(i * LANES, LANES)
            call[j, sl] = call[j, sl] * MAXLEN + (iota + (i * LANES + posbase))
        return carry

    lax.fori_loop(0, nchunk_w, cidx_body, 0)

    for q in range(SUB):
        for i in range(SUBROWS // LANES):
            ident[q, pl.ds(i * LANES, LANES)] = iota + (q * SUBROWS + i * LANES)

    def fire_gathers(c, b):
        cpt = pltpu.async_copy(tok_hbm.at[xall.at[c]], toks[b], semts[b])
        cpp = pltpu.async_copy(psum_shr.at[call.at[c]], pss[b], semps[b])
        return cpt, cpp

    def turn(c, b, refire):
        pltpu.make_async_copy(tok_hbm.at[xall.at[c]], toks[b], semts[b]).wait()
        pltpu.make_async_copy(psum_shr.at[call.at[c]], pss[b], semps[b]).wait()
        rowbase = (w * nchunk_w + c) * CHUNK
        out_handles = []
        for q in range(SUB):
            @plsc.parallel_loop(q * SUBROWS, (q + 1) * SUBROWS, unroll=4)
            def add_body(r):
                for j in range(D // LANES):
                    sl = pl.ds(j * LANES, LANES)
                    plsc.addupdate(pss[b].at[r, sl], toks[b][r, sl])

            out_handles.append(pltpu.async_copy(
                pss[b].at[pl.ds(q * SUBROWS, SUBROWS)],
                out_hbm.at[pl.ds(rowbase + q * SUBROWS, SUBROWS)],
                semos[b]))
        if refire:
            pltpu.async_copy(tok_hbm.at[xall.at[c + 2]], toks[b], semts[b])
        for h in out_handles:
            h.wait()
        if refire:
            pltpu.async_copy(psum_shr.at[call.at[c + 2]], pss[b], semps[b])

    fire_gathers(0, 0)
    fire_gathers(1, 1)

    def main_body(cc, carry):
        for b in range(2):
            turn(2 * cc + b, b, refire=True)
        return carry

    lax.fori_loop(0, nchunk_w // 2 - 1, main_body, 0)
    turn(nchunk_w - 2, 0, refire=False)
    turn(nchunk_w - 1, 1, refire=False)


def _sc_gather(tok_embed, pos_embed, seg_embed, x_blk, seg_blk):
    nblk = x_blk.shape[0]
    rows = nblk * CHUNK
    fn = functools.partial(
        pl.kernel,
        out_type=jax.ShapeDtypeStruct((rows, D), jnp.float32),
        mesh=plsc.VectorSubcoreMesh(core_axis_name="c", subcore_axis_name="s"),
        scratch_types=[
            pltpu.VMEM_SHARED((MAXLEN * NSEG, D), jnp.float32),
            pltpu.VMEM((nblk // NW, CHUNK), jnp.int32),
            pltpu.VMEM((nblk // NW, CHUNK), jnp.int32),
            pltpu.VMEM((SUB, SUBROWS), jnp.int32),
            pltpu.VMEM((CHUNK, D), jnp.float32),
            pltpu.VMEM((CHUNK, D), jnp.float32),
            pltpu.VMEM((CHUNK, D), jnp.float32),
            pltpu.VMEM((CHUNK, D), jnp.float32),
            pltpu.SemaphoreType.DMA,
            pltpu.SemaphoreType.DMA,
            pltpu.SemaphoreType.DMA,
            pltpu.SemaphoreType.DMA,
            pltpu.SemaphoreType.DMA,
            pltpu.SemaphoreType.DMA,
        ],
    )(_sc_body)
    return fn(tok_embed, pos_embed, seg_embed, x_blk, seg_blk)


def kernel(x, seg, tok_embed, pos_embed, seg_embed):
    batch, seqlen = x.shape
    x_blk = x.reshape(-1, CHUNK).astype(jnp.int32)
    seg_blk = seg.reshape(-1, CHUNK).astype(jnp.int32)
    out = _sc_gather(tok_embed, pos_embed, seg_embed, x_blk, seg_blk)
    return out.reshape(batch, seqlen, D)


# speedup vs baseline: 23.5152x; 1.0051x over previous
import functools

import jax
import jax.numpy as jnp
from jax import lax
from jax.experimental import pallas as pl
from jax.experimental.pallas import tpu as pltpu
from jax.experimental.pallas import tpu_sc as plsc

D = 128
MAXLEN = 512
NSEG = 2
NC = 2
NS = 16
NW = NC * NS
CHUNK = 64
LANES = 16
NBUF = 3


def _sc_body(tok_hbm, pos_hbm, seg_emb_hbm, x_hbm, seg_hbm, out_hbm,
             psum_shr, xall, call, tok0, ps0, tok1, ps1, tok2, ps2,
             semt0, semp0, semo0, semt1, semp1, semo1, semt2, semp2, semo2):
    nchunk_w = xall.shape[0]
    w = lax.axis_index("s") * NC + lax.axis_index("c")
    iota = lax.iota(jnp.int32, LANES)
    toks = (tok0, tok1, tok2)
    pss = (ps0, ps1, ps2)
    semts = (semt0, semt1, semt2)
    semps = (semp0, semp1, semp2)
    semos = (semo0, semo1, semo2)

    sid = lax.axis_index("s")
    prows = (MAXLEN * NSEG) // NS
    g = sid // (MAXLEN // prows)
    s0 = lax.rem(sid * prows, MAXLEN)
    pltpu.sync_copy(seg_emb_hbm, tok0.at[pl.ds(0, NSEG)])
    pltpu.sync_copy(pos_hbm.at[pl.ds(s0, prows)], ps0.at[pl.ds(0, prows)])

    def prow_body(r, carry):
        for j in range(D // LANES):
            sl = pl.ds(j * LANES, LANES)
            ps0[r, sl] = ps0[r, sl] + tok0[g, sl]
        return carry

    lax.fori_loop(0, prows, prow_body, 0, unroll=2)
    pltpu.sync_copy(ps0.at[pl.ds(0, prows)],
                    psum_shr.at[pl.ds(sid * prows, prows)])
    plsc.subcore_barrier()

    pltpu.sync_copy(x_hbm.at[pl.ds(w * nchunk_w, nchunk_w)], xall)
    pltpu.sync_copy(seg_hbm.at[pl.ds(w * nchunk_w, nchunk_w)], call)

    def cidx_body(j, carry):
        posbase = lax.rem(j, MAXLEN // CHUNK) * CHUNK
        for i in range(CHUNK // LANES):
            sl = pl.ds(i * LANES, LANES)
            call[j, sl] = call[j, sl] * MAXLEN + (iota + (i * LANES + posbase))
        return carry

    lax.fori_loop(0, nchunk_w, cidx_body, 0)

    def fire_tok(c, b):
        pltpu.async_copy(tok_hbm.at[xall.at[c]], toks[b], semts[b])

    def fire_ps(c, b):
        pltpu.async_copy(psum_shr.at[call.at[c]], pss[b], semps[b])

    def out_copy(c, b):
        rowbase = (w * nchunk_w + c) * CHUNK
        return pltpu.make_async_copy(
            pss[b], out_hbm.at[pl.ds(rowbase, CHUNK)], semos[b])

    def turn(c, b, bprev, waitprev, refire_tok, refire_ps):
        pltpu.make_async_copy(tok_hbm.at[xall.at[c]], toks[b], semts[b]).wait()
        pltpu.make_async_copy(psum_shr.at[call.at[c]], pss[b], semps[b]).wait()

        @plsc.parallel_loop(0, CHUNK, unroll=4)
        def add_body(r):
            for j in range(D // LANES):
                sl = pl.ds(j * LANES, LANES)
                plsc.addupdate(pss[b].at[r, sl], toks[b][r, sl])

        out_copy(c, b).start()
        if refire_tok:
            fire_tok(c + NBUF, b)
        if waitprev:
            out_copy(c - 1, bprev).wait()
            if refire_ps:
                fire_ps(c + NBUF - 1, bprev)

    for c in range(NBUF):
        fire_tok(c, c)
        fire_ps(c, c)

    turn(0, 0, NBUF - 1, False, True, False)

    n_uni = nchunk_w - 1 - NBUF
    assert n_uni % NBUF == 0

    def main_body(cc, carry):
        for db in range(NBUF):
            c = NBUF * cc + 1 + db
            turn(c, (1 + db) % NBUF, db, True, True, True)
        return carry

    lax.fori_loop(0, n_uni // NBUF, main_body, 0)

    c0 = nchunk_w - NBUF
    turn(c0, c0 % NBUF, (c0 - 1) % NBUF, True, False, True)
    turn(c0 + 1, (c0 + 1) % NBUF, c0 % NBUF, True, False, False)
    turn(c0 + 2, (c0 + 2) % NBUF, (c0 + 1) % NBUF, True, False, False)
    out_copy(nchunk_w - 1, (nchunk_w - 1) % NBUF).wait()


def _sc_gather(tok_embed, pos_embed, seg_embed, x_blk, seg_blk):
    nblk = x_blk.shape[0]
    rows = nblk * CHUNK
    fn = functools.partial(
        pl.kernel,
        out_type=jax.ShapeDtypeStruct((rows, D), jnp.float32),
        mesh=plsc.VectorSubcoreMesh(core_axis_name="c", subcore_axis_name="s"),
        scratch_types=[
            pltpu.VMEM_SHARED((MAXLEN * NSEG, D), jnp.float32),
            pltpu.VMEM((nblk // NW, CHUNK), jnp.int32),
            pltpu.VMEM((nblk // NW, CHUNK), jnp.int32),
            pltpu.VMEM((CHUNK, D), jnp.float32),
            pltpu.VMEM((CHUNK, D), jnp.float32),
            pltpu.VMEM((CHUNK, D), jnp.float32),
            pltpu.VMEM((CHUNK, D), jnp.float32),
            pltpu.VMEM((CHUNK, D), jnp.float32),
            pltpu.VMEM((CHUNK, D), jnp.float32),
            pltpu.SemaphoreType.DMA,
            pltpu.SemaphoreType.DMA,
            pltpu.SemaphoreType.DMA,
            pltpu.SemaphoreType.DMA,
            pltpu.SemaphoreType.DMA,
            pltpu.SemaphoreType.DMA,
            pltpu.SemaphoreType.DMA,
            pltpu.SemaphoreType.DMA,
            pltpu.SemaphoreType.DMA,
        ],
    )(_sc_body)
    return fn(tok_embed, pos_embed, seg_embed, x_blk, seg_blk)


def kernel(x, seg, tok_embed, pos_embed, seg_embed):
    batch, seqlen = x.shape
    x_blk = x.reshape(-1, CHUNK).astype(jnp.int32)
    seg_blk = seg.reshape(-1, CHUNK).astype(jnp.int32)
    out = _sc_gather(tok_embed, pos_embed, seg_embed, x_blk, seg_blk)
    return out.reshape(batch, seqlen, D)


# speedup vs baseline: 23.5841x; 1.0029x over previous
import functools

import jax
import jax.numpy as jnp
from jax import lax
from jax.experimental import pallas as pl
from jax.experimental.pallas import tpu as pltpu
from jax.experimental.pallas import tpu_sc as plsc

D = 128
MAXLEN = 512
NSEG = 2
NC = 2
NS = 16
NW = NC * NS
CHUNK = 64
LANES = 16
NBUF = 3


def _sc_body(tok_hbm, pos_hbm, seg_emb_hbm, x_hbm, seg_hbm, out_hbm,
             psum_shr, xall, call, tok0, ps0, tok1, ps1, tok2, ps2,
             semt0, semp0, semo0, semt1, semp1, semo1, semt2, semp2, semo2):
    nchunk_w = xall.shape[0]
    w = lax.axis_index("s") * NC + lax.axis_index("c")
    iota = lax.iota(jnp.int32, LANES)
    toks = (tok0, tok1, tok2)
    pss = (ps0, ps1, ps2)
    semts = (semt0, semt1, semt2)
    semps = (semp0, semp1, semp2)
    semos = (semo0, semo1, semo2)

    cpx = pltpu.make_async_copy(
        x_hbm.at[pl.ds(w * xall.shape[0], xall.shape[0])], xall, semt1)
    cpc = pltpu.make_async_copy(
        seg_hbm.at[pl.ds(w * call.shape[0], call.shape[0])], call, semp1)
    cpx.start()
    cpc.start()

    sid = lax.axis_index("s")
    prows = (MAXLEN * NSEG) // NS
    g = sid // (MAXLEN // prows)
    s0 = lax.rem(sid * prows, MAXLEN)
    pltpu.sync_copy(seg_emb_hbm, tok0.at[pl.ds(0, NSEG)])
    pltpu.sync_copy(pos_hbm.at[pl.ds(s0, prows)], ps0.at[pl.ds(0, prows)])

    def prow_body(r, carry):
        for j in range(D // LANES):
            sl = pl.ds(j * LANES, LANES)
            ps0[r, sl] = ps0[r, sl] + tok0[g, sl]
        return carry

    lax.fori_loop(0, prows, prow_body, 0, unroll=2)
    pltpu.sync_copy(ps0.at[pl.ds(0, prows)],
                    psum_shr.at[pl.ds(sid * prows, prows)])
    plsc.subcore_barrier()

    cpx.wait()
    cpc.wait()

    def cidx_body(j, carry):
        posbase = lax.rem(j, MAXLEN // CHUNK) * CHUNK
        for i in range(CHUNK // LANES):
            sl = pl.ds(i * LANES, LANES)
            call[j, sl] = call[j, sl] * MAXLEN + (iota + (i * LANES + posbase))
        return carry

    lax.fori_loop(0, nchunk_w, cidx_body, 0)

    def fire_tok(c, b):
        pltpu.async_copy(tok_hbm.at[xall.at[c]], toks[b], semts[b])

    def fire_ps(c, b):
        pltpu.async_copy(psum_shr.at[call.at[c]], pss[b], semps[b])

    def out_copy(c, b):
        rowbase = (w * nchunk_w + c) * CHUNK
        return pltpu.make_async_copy(
            pss[b], out_hbm.at[pl.ds(rowbase, CHUNK)], semos[b])

    def turn(c, b, bprev, waitprev, refire_tok, refire_ps):
        pltpu.make_async_copy(tok_hbm.at[xall.at[c]], toks[b], semts[b]).wait()
        pltpu.make_async_copy(psum_shr.at[call.at[c]], pss[b], semps[b]).wait()

        @plsc.parallel_loop(0, CHUNK, unroll=8)
        def add_body(r):
            for j in range(D // LANES):
                sl = pl.ds(j * LANES, LANES)
                plsc.addupdate(pss[b].at[r, sl], toks[b][r, sl])

        out_copy(c, b).start()
        if refire_tok:
            fire_tok(c + NBUF, b)
        if waitprev:
            out_copy(c - 1, bprev).wait()
            if refire_ps:
                fire_ps(c + NBUF - 1, bprev)

    for c in range(NBUF):
        fire_tok(c, c)
        fire_ps(c, c)

    turn(0, 0, NBUF - 1, False, True, False)

    n_uni = nchunk_w - 1 - NBUF
    assert n_uni % NBUF == 0

    def main_body(cc, carry):
        for db in range(NBUF):
            c = NBUF * cc + 1 + db
            turn(c, (1 + db) % NBUF, db, True, True, True)
        return carry

    lax.fori_loop(0, n_uni // NBUF, main_body, 0)

    c0 = nchunk_w - NBUF
    turn(c0, c0 % NBUF, (c0 - 1) % NBUF, True, False, True)
    turn(c0 + 1, (c0 + 1) % NBUF, c0 % NBUF, True, False, False)
    turn(c0 + 2, (c0 + 2) % NBUF, (c0 + 1) % NBUF, True, False, False)
    out_copy(nchunk_w - 1, (nchunk_w - 1) % NBUF).wait()


def _sc_gather(tok_embed, pos_embed, seg_embed, x_blk, seg_blk):
    nblk = x_blk.shape[0]
    rows = nblk * CHUNK
    fn = functools.partial(
        pl.kernel,
        out_type=jax.ShapeDtypeStruct((rows, D), jnp.float32),
        mesh=plsc.VectorSubcoreMesh(core_axis_name="c", subcore_axis_name="s"),
        scratch_types=[
            pltpu.VMEM_SHARED((MAXLEN * NSEG, D), jnp.float32),
            pltpu.VMEM((nblk // NW, CHUNK), jnp.int32),
            pltpu.VMEM((nblk // NW, CHUNK), jnp.int32),
            pltpu.VMEM((CHUNK, D), jnp.float32),
            pltpu.VMEM((CHUNK, D), jnp.float32),
            pltpu.VMEM((CHUNK, D), jnp.float32),
            pltpu.VMEM((CHUNK, D), jnp.float32),
            pltpu.VMEM((CHUNK, D), jnp.float32),
            pltpu.VMEM((CHUNK, D), jnp.float32),
            pltpu.SemaphoreType.DMA,
            pltpu.SemaphoreType.DMA,
            pltpu.SemaphoreType.DMA,
            pltpu.SemaphoreType.DMA,
            pltpu.SemaphoreType.DMA,
            pltpu.SemaphoreType.DMA,
            pltpu.SemaphoreType.DMA,
            pltpu.SemaphoreType.DMA,
            pltpu.SemaphoreType.DMA,
        ],
    )(_sc_body)
    return fn(tok_embed, pos_embed, seg_embed, x_blk, seg_blk)


def kernel(x, seg, tok_embed, pos_embed, seg_embed):
    batch, seqlen = x.shape
    x_blk = x.reshape(-1, CHUNK).astype(jnp.int32)
    seg_blk = seg.reshape(-1, CHUNK).astype(jnp.int32)
    out = _sc_gather(tok_embed, pos_embed, seg_embed, x_blk, seg_blk)
    return out.reshape(batch, seqlen, D)
